# Initial kernel scaffold; baseline (speedup 1.0000x reference)
import jax
import jax.numpy as jnp
from jax.experimental import pallas as pl


def kernel(flow_x, flow_y):
    raise NotImplementedError("write your pallas kernel here")


# speedup vs baseline: 2.0793x; 2.0793x over previous
import functools

import jax
import jax.numpy as jnp
from jax import lax
from jax.experimental import pallas as pl
from jax.experimental.pallas import tpu as pltpu
from jax.experimental.pallas import tpu_sc as plsc

B = 8
P = 4
H = 480
W = 640
N = H * W
NPAIR = B * (P - 1)
SPAT_W = 0.001
TEMP_W = 0.001

NW = 32
ROWS_PER_W = H // NW
PX_PER_W = ROWS_PER_W * W
ROWS_PER_CHUNK = 5
CHUNK = ROWS_PER_CHUNK * W
NCHUNK = ROWS_PER_W // ROWS_PER_CHUNK
FSLICE = 128
NFIRE = CHUNK // FSLICE
CVREG = W // 16


def _spatial_body(fx_ref, fy_ref, out_ref):
    fx = fx_ref[0, 0]
    fy = fy_ref[0, 0]

    def charb(v):
        return jnp.sqrt(v * v + 1e-6)

    s_dx = jnp.sum(charb(fx[:, :-1] - fx[:, 1:]) + charb(fy[:, :-1] - fy[:, 1:]))
    s_dy = jnp.sum(charb(fx[:-1, :] - fx[1:, :]) + charb(fy[:-1, :] - fy[1:, :]))
    s_dr = jnp.sum(charb(fx[:-1, :-1] - fx[1:, 1:]) + charb(fy[:-1, :-1] - fy[1:, 1:]))
    s_ur = jnp.sum(charb(fx[1:, :-1] - fx[:-1, 1:]) + charb(fy[1:, :-1] - fy[:-1, 1:]))
    out_ref[0, 0, :] = jnp.stack([s_dx, s_dy, s_dr, s_ur])


_spatial_call = pl.pallas_call(
    _spatial_body,
    grid=(B * P,),
    in_specs=[
        pl.BlockSpec((1, 1, H, W), lambda i: (i // P, i % P, 0, 0)),
        pl.BlockSpec((1, 1, H, W), lambda i: (i // P, i % P, 0, 0)),
    ],
    out_specs=pl.BlockSpec((1, 1, 4), lambda i: (i, 0, 0)),
    out_shape=jax.ShapeDtypeStruct((B * P, 1, 4), jnp.float32),
)


def _sqrt16(v):
    i = lax.bitcast_convert_type(v, jnp.int32)
    i = jnp.full((16,), 0x5F3759DF, jnp.int32) - (i >> 1)
    r = lax.bitcast_convert_type(i, jnp.float32)
    half = v * 0.5
    r = r * (1.5 - half * r * r)
    r = r * (1.5 - half * r * r)
    return v * r


def _unpack_fx(word_f32):
    w = lax.bitcast_convert_type(word_f32, jnp.int32)
    return lax.bitcast_convert_type(w << 16, jnp.float32)


def _unpack_fy(word_f32):
    w = lax.bitcast_convert_type(word_f32, jnp.int32)
    return lax.bitcast_convert_type(w & jnp.int32(-65536), jnp.float32)


_sc_mesh = plsc.VectorSubcoreMesh(core_axis_name="c", subcore_axis_name="s")


@functools.partial(
    pl.kernel,
    mesh=_sc_mesh,
    out_type=jax.ShapeDtypeStruct((NW, NPAIR, 2, 16), jnp.float32),
    scratch_types=[
        pltpu.VMEM((PX_PER_W,), jnp.float32),
        pltpu.VMEM((PX_PER_W,), jnp.float32),
        pltpu.VMEM((CHUNK,), jnp.int32),
        pltpu.VMEM((CHUNK,), jnp.int32),
        pltpu.VMEM((CHUNK,), jnp.int32),
        pltpu.VMEM((CHUNK,), jnp.int32),
        pltpu.VMEM((CHUNK,), jnp.float32),
        pltpu.VMEM((CHUNK,), jnp.float32),
        pltpu.VMEM((CHUNK,), jnp.float32),
        pltpu.VMEM((CHUNK,), jnp.float32),
        pltpu.VMEM((NPAIR, 2, 16), jnp.float32),
        pltpu.SemaphoreType.DMA,
    ],
)
def _temporal_sc(fx_hbm, fy_hbm, tab_hbm, out_hbm,
                 fxs, fys, i00, i01, i10, i11, g00, g01, g10, g11, outv, sem):
    wid = lax.axis_index("s") * 2 + lax.axis_index("c")
    base_px = wid * PX_PER_W
    base_row = wid * ROWS_PER_W
    lane = lax.iota(jnp.int32, 16)
    lane_f = lane.astype(jnp.float32)

    def pair_body(b, jj):
        pair = b * 3 + jj
        img = b * P + jj
        pltpu.sync_copy(fx_hbm.at[pl.ds(img * N + base_px, PX_PER_W)], fxs)
        pltpu.sync_copy(fy_hbm.at[pl.ds(img * N + base_px, PX_PER_W)], fys)
        tab_base = pair * N

        def chunk_accum(ci, accs):
            def idx_row(r, _):
                y = base_row + ci * ROWS_PER_CHUNK + r
                yf = y.astype(jnp.float32)

                def idx_col(c, _):
                    o = ((ci * ROWS_PER_CHUNK + r) * CVREG + c) * 16
                    oc = (r * CVREG + c) * 16
                    fxv = fxs[pl.ds(o, 16)]
                    fyv = fys[pl.ds(o, 16)]
                    xf = (c * 16).astype(jnp.float32) + lane_f
                    ycl = jnp.minimum(jnp.maximum(yf + fyv, 0.0), H - 1.0)
                    xcl = jnp.minimum(jnp.maximum(xf + fxv, 0.0), W - 1.0)
                    y0 = ycl.astype(jnp.int32)
                    x0 = xcl.astype(jnp.int32)
                    r00 = tab_base + y0 * W + x0
                    dx1 = jnp.where(x0 < W - 1, 1, 0)
                    r10 = jnp.where(y0 < H - 1, r00 + W, r00)
                    i00[pl.ds(oc, 16)] = r00
                    i01[pl.ds(oc, 16)] = r00 + dx1
                    i10[pl.ds(oc, 16)] = r10
                    i11[pl.ds(oc, 16)] = r10 + dx1
                    return _

                lax.fori_loop(0, CVREG, idx_col, 0)
                return _

            lax.fori_loop(0, ROWS_PER_CHUNK, idx_row, 0)

            cps = []
            for f in range(NFIRE):
                sl = pl.ds(f * FSLICE, FSLICE)
                cps.append(pltpu.async_copy(tab_hbm.at[i00.at[sl]], g00.at[sl], sem))
                cps.append(pltpu.async_copy(tab_hbm.at[i01.at[sl]], g01.at[sl], sem))
                cps.append(pltpu.async_copy(tab_hbm.at[i10.at[sl]], g10.at[sl], sem))
                cps.append(pltpu.async_copy(tab_hbm.at[i11.at[sl]], g11.at[sl], sem))
            for cp in cps:
                cp.wait()

            def comb_row(r, accs):
                y = base_row + ci * ROWS_PER_CHUNK + r
                yf = y.astype(jnp.float32)

                def comb_col(c, accs):
                    acc_dt, acc_m = accs
                    o = ((ci * ROWS_PER_CHUNK + r) * CVREG + c) * 16
                    oc = (r * CVREG + c) * 16
                    fxv = fxs[pl.ds(o, 16)]
                    fyv = fys[pl.ds(o, 16)]
                    xf = (c * 16).astype(jnp.float32) + lane_f
                    wy = yf + fyv
                    wx = xf + fxv
                    ycl = jnp.minimum(jnp.maximum(wy, 0.0), H - 1.0)
                    xcl = jnp.minimum(jnp.maximum(wx, 0.0), W - 1.0)
                    y0f = ycl.astype(jnp.int32).astype(jnp.float32)
                    x0f = xcl.astype(jnp.int32).astype(jnp.float32)
                    wy1 = ycl - y0f
                    wy0 = 1.0 - wy1
                    wx1 = xcl - x0f
                    wx0 = 1.0 - wx1
                    v00 = g00[pl.ds(oc, 16)]
                    v01 = g01[pl.ds(oc, 16)]
                    v10 = g10[pl.ds(oc, 16)]
                    v11 = g11[pl.ds(oc, 16)]
                    w00 = wy0 * wx0
                    w01 = wy0 * wx1
                    w10 = wy1 * wx0
                    w11 = wy1 * wx1
                    wfx = (w00 * _unpack_fx(v00) + w01 * _unpack_fx(v01)
                           + w10 * _unpack_fx(v10) + w11 * _unpack_fx(v11))
                    wfy = (w00 * _unpack_fy(v00) + w01 * _unpack_fy(v01)
                           + w10 * _unpack_fy(v10) + w11 * _unpack_fy(v11))
                    dy = fyv - wfy
                    dx = fxv - wfx
                    dt = _sqrt16(dy * dy + 1e-9) + _sqrt16(dx * dx + 1e-9)
                    inb = ((wy >= 0.0) & (wy <= H - 1.0)
                           & (wx >= 0.0) & (wx <= W - 1.0))
                    m = jnp.where(inb, 1.0, 0.0)
                    return acc_dt + dt * m, acc_m + m

                return lax.fori_loop(0, CVREG, comb_col, accs)

            return lax.fori_loop(0, ROWS_PER_CHUNK, comb_row, accs)

        zero = jnp.zeros((16,), jnp.float32)
        acc_dt, acc_m = lax.fori_loop(0, NCHUNK, chunk_accum, (zero, zero))
        outv[pair, 0] = acc_dt
        outv[pair, 1] = acc_m

    def b_body(b, _):
        def j_body(jj, _):
            pair_body(b, jj)
            return _
        lax.fori_loop(0, P - 1, j_body, 0)
        return _

    lax.fori_loop(0, B, b_body, 0)
    pltpu.sync_copy(outv, out_hbm.at[wid])


def kernel(flow_x, flow_y):
    sp = _spatial_call(flow_x, flow_y).reshape(B, P, 4)

    fxi = flow_x[:, 1:].reshape(NPAIR * N)
    fyi = flow_y[:, 1:].reshape(NPAIR * N)
    fx16 = lax.bitcast_convert_type(fxi.astype(jnp.bfloat16), jnp.uint16)
    fy16 = lax.bitcast_convert_type(fyi.astype(jnp.bfloat16), jnp.uint16)
    word = (fy16.astype(jnp.uint32) << 16) | fx16.astype(jnp.uint32)
    tab = lax.bitcast_convert_type(word, jnp.float32)

    part = _temporal_sc(flow_x.reshape(B * P * N), flow_y.reshape(B * P * N), tab)

    d_dx = float(H * (W - 1))
    d_dy = float((H - 1) * W)
    d_di = float((H - 1) * (W - 1))
    per_b = (sp[:, :, 0].sum(1) / d_dx + sp[:, :, 1].sum(1) / d_dy
             + sp[:, :, 2].sum(1) / d_di + sp[:, :, 3].sum(1) / d_di) / P
    spatial = SPAT_W * jnp.sum(per_b / 4.0)

    q = part.sum(axis=(0, 3))
    dt = q[:, 0].reshape(B, P - 1)
    mm = q[:, 1].reshape(B, P - 1)
    per_b_t = (dt / (mm + 1e-9)).sum(axis=1) / (P - 1)
    temporal = TEMP_W * jnp.sum(per_b_t)

    return spatial + temporal


# speedup vs baseline: 2.3065x; 1.1092x over previous
import functools

import jax
import jax.numpy as jnp
from jax import lax
from jax.experimental import pallas as pl
from jax.experimental.pallas import tpu as pltpu
from jax.experimental.pallas import tpu_sc as plsc

B = 8
P = 4
H = 480
W = 640
N = H * W
NPAIR = B * (P - 1)
SPAT_W = 0.001
TEMP_W = 0.001

NW = 32
ROWS_PER_W = H // NW
PX_PER_W = ROWS_PER_W * W
ROWS_PER_CHUNK = 5
CHUNK = ROWS_PER_CHUNK * W
NCHUNK = ROWS_PER_W // ROWS_PER_CHUNK
FSLICE = 128
NFIRE = CHUNK // FSLICE
CVREG = W // 16


def _spatial_body(fx_ref, fy_ref, out_ref):
    fx = fx_ref[0, 0]
    fy = fy_ref[0, 0]

    def charb(v):
        return jnp.sqrt(v * v + 1e-6)

    s_dx = jnp.sum(charb(fx[:, :-1] - fx[:, 1:]) + charb(fy[:, :-1] - fy[:, 1:]))
    s_dy = jnp.sum(charb(fx[:-1, :] - fx[1:, :]) + charb(fy[:-1, :] - fy[1:, :]))
    s_dr = jnp.sum(charb(fx[:-1, :-1] - fx[1:, 1:]) + charb(fy[:-1, :-1] - fy[1:, 1:]))
    s_ur = jnp.sum(charb(fx[1:, :-1] - fx[:-1, 1:]) + charb(fy[1:, :-1] - fy[:-1, 1:]))
    out_ref[0, 0, :] = jnp.stack([s_dx, s_dy, s_dr, s_ur])


_spatial_call = pl.pallas_call(
    _spatial_body,
    grid=(B * P,),
    in_specs=[
        pl.BlockSpec((1, 1, H, W), lambda i: (i // P, i % P, 0, 0)),
        pl.BlockSpec((1, 1, H, W), lambda i: (i // P, i % P, 0, 0)),
    ],
    out_specs=pl.BlockSpec((1, 1, 4), lambda i: (i, 0, 0)),
    out_shape=jax.ShapeDtypeStruct((B * P, 1, 4), jnp.float32),
)


def _sqrt16(v):
    i = lax.bitcast_convert_type(v, jnp.int32)
    i = jnp.full((16,), 0x5F3759DF, jnp.int32) - (i >> 1)
    r = lax.bitcast_convert_type(i, jnp.float32)
    half = v * 0.5
    r = r * (1.5 - half * r * r)
    r = r * (1.5 - half * r * r)
    return v * r


def _unpack_fx(word_f32):
    w = lax.bitcast_convert_type(word_f32, jnp.int32)
    return lax.bitcast_convert_type(w << 16, jnp.float32)


def _unpack_fy(word_f32):
    w = lax.bitcast_convert_type(word_f32, jnp.int32)
    return lax.bitcast_convert_type(w & jnp.int32(-65536), jnp.float32)


_sc_mesh = plsc.VectorSubcoreMesh(core_axis_name="c", subcore_axis_name="s")


@functools.partial(
    pl.kernel,
    mesh=_sc_mesh,
    out_type=jax.ShapeDtypeStruct((NW, NPAIR, 2, 16), jnp.float32),
    scratch_types=[
        pltpu.VMEM((PX_PER_W,), jnp.float32),
        pltpu.VMEM((PX_PER_W,), jnp.float32),
        pltpu.VMEM((2, CHUNK), jnp.int32),
        pltpu.VMEM((2, CHUNK), jnp.int32),
        pltpu.VMEM((2, CHUNK), jnp.int32),
        pltpu.VMEM((2, CHUNK), jnp.int32),
        pltpu.VMEM((2, CHUNK), jnp.float32),
        pltpu.VMEM((2, CHUNK), jnp.float32),
        pltpu.VMEM((2, CHUNK), jnp.float32),
        pltpu.VMEM((2, CHUNK), jnp.float32),
        pltpu.VMEM((NPAIR, 2, 16), jnp.float32),
        pltpu.SemaphoreType.DMA,
        pltpu.SemaphoreType.DMA,
    ],
)
def _temporal_sc(fx_hbm, fy_hbm, tab_hbm, out_hbm,
                 fxs, fys, i00, i01, i10, i11, g00, g01, g10, g11, outv,
                 sem0, sem1):
    wid = lax.axis_index("s") * 2 + lax.axis_index("c")
    base_px = wid * PX_PER_W
    base_row = wid * ROWS_PER_W
    lane = lax.iota(jnp.int32, 16)
    lane_f = lane.astype(jnp.float32)

    ROW_FIRES = W // FSLICE

    def pair_body(b, jj):
        pair = b * 3 + jj
        img = b * P + jj
        pltpu.sync_copy(fx_hbm.at[pl.ds(img * N + base_px, PX_PER_W)], fxs)
        pltpu.sync_copy(fy_hbm.at[pl.ds(img * N + base_px, PX_PER_W)], fys)
        tab_base = pair * N
        ibufs = (i00, i01, i10, i11)
        gbufs = (g00, g01, g10, g11)
        sems = (sem0, sem1)

        def idx_and_fire(ci, s):
            sem = sems[s]
            for r in range(ROWS_PER_CHUNK):
                y = base_row + ci * ROWS_PER_CHUNK + r
                yf = y.astype(jnp.float32)

                def idx_col(c, _):
                    o = ((ci * ROWS_PER_CHUNK + r) * CVREG + c) * 16
                    oc = (r * CVREG + c) * 16
                    fxv = fxs[pl.ds(o, 16)]
                    fyv = fys[pl.ds(o, 16)]
                    xf = (c * 16).astype(jnp.float32) + lane_f
                    ycl = jnp.minimum(jnp.maximum(yf + fyv, 0.0), H - 1.0)
                    xcl = jnp.minimum(jnp.maximum(xf + fxv, 0.0), W - 1.0)
                    y0 = ycl.astype(jnp.int32)
                    x0 = xcl.astype(jnp.int32)
                    r00 = tab_base + y0 * W + x0
                    dx1 = jnp.where(x0 < W - 1, 1, 0)
                    r10 = jnp.where(y0 < H - 1, r00 + W, r00)
                    i00[s, pl.ds(oc, 16)] = r00
                    i01[s, pl.ds(oc, 16)] = r00 + dx1
                    i10[s, pl.ds(oc, 16)] = r10
                    i11[s, pl.ds(oc, 16)] = r10 + dx1
                    return _

                lax.fori_loop(0, CVREG, idx_col, 0)
                for f in range(ROW_FIRES):
                    sl = pl.ds((r * ROW_FIRES + f) * FSLICE, FSLICE)
                    for ib, gb in zip(ibufs, gbufs):
                        pltpu.async_copy(tab_hbm.at[ib.at[s, sl]], gb.at[s, sl], sem)

        def drain(s):
            for gb in gbufs:
                pltpu.make_async_copy(tab_hbm.at[pl.ds(0, CHUNK)], gb.at[s], sems[s]).wait()

        def combine(ci, s, accs):
            def comb_row(r, accs):
                y = base_row + ci * ROWS_PER_CHUNK + r
                yf = y.astype(jnp.float32)

                def comb_col(c, accs):
                    acc_dt, acc_m = accs
                    o = ((ci * ROWS_PER_CHUNK + r) * CVREG + c) * 16
                    oc = (r * CVREG + c) * 16
                    fxv = fxs[pl.ds(o, 16)]
                    fyv = fys[pl.ds(o, 16)]
                    xf = (c * 16).astype(jnp.float32) + lane_f
                    wy = yf + fyv
                    wx = xf + fxv
                    ycl = jnp.minimum(jnp.maximum(wy, 0.0), H - 1.0)
                    xcl = jnp.minimum(jnp.maximum(wx, 0.0), W - 1.0)
                    y0f = ycl.astype(jnp.int32).astype(jnp.float32)
                    x0f = xcl.astype(jnp.int32).astype(jnp.float32)
                    wy1 = ycl - y0f
                    wy0 = 1.0 - wy1
                    wx1 = xcl - x0f
                    wx0 = 1.0 - wx1
                    v00 = g00[s, pl.ds(oc, 16)]
                    v01 = g01[s, pl.ds(oc, 16)]
                    v10 = g10[s, pl.ds(oc, 16)]
                    v11 = g11[s, pl.ds(oc, 16)]
                    w00 = wy0 * wx0
                    w01 = wy0 * wx1
                    w10 = wy1 * wx0
                    w11 = wy1 * wx1
                    wfx = (w00 * _unpack_fx(v00) + w01 * _unpack_fx(v01)
                           + w10 * _unpack_fx(v10) + w11 * _unpack_fx(v11))
                    wfy = (w00 * _unpack_fy(v00) + w01 * _unpack_fy(v01)
                           + w10 * _unpack_fy(v10) + w11 * _unpack_fy(v11))
                    dy = fyv - wfy
                    dx = fxv - wfx
                    dt = _sqrt16(dy * dy + 1e-9) + _sqrt16(dx * dx + 1e-9)
                    inb = ((wy >= 0.0) & (wy <= H - 1.0)
                           & (wx >= 0.0) & (wx <= W - 1.0))
                    m = jnp.where(inb, 1.0, 0.0)
                    return acc_dt + dt * m, acc_m + m

                return lax.fori_loop(0, CVREG, comb_col, accs)

            return lax.fori_loop(0, ROWS_PER_CHUNK, comb_row, accs)

        zero = jnp.zeros((16,), jnp.float32)
        accs = (zero, zero)
        idx_and_fire(0, 0)
        for ci in range(1, NCHUNK):
            idx_and_fire(ci, ci % 2)
            drain((ci - 1) % 2)
            accs = combine(ci - 1, (ci - 1) % 2, accs)
        drain((NCHUNK - 1) % 2)
        acc_dt, acc_m = combine(NCHUNK - 1, (NCHUNK - 1) % 2, accs)
        outv[pair, 0] = acc_dt
        outv[pair, 1] = acc_m

    def b_body(b, _):
        def j_body(jj, _):
            pair_body(b, jj)
            return _
        lax.fori_loop(0, P - 1, j_body, 0)
        return _

    lax.fori_loop(0, B, b_body, 0)
    pltpu.sync_copy(outv, out_hbm.at[wid])


def kernel(flow_x, flow_y):
    sp = _spatial_call(flow_x, flow_y).reshape(B, P, 4)

    fxi = flow_x[:, 1:].reshape(NPAIR * N)
    fyi = flow_y[:, 1:].reshape(NPAIR * N)
    fx16 = lax.bitcast_convert_type(fxi.astype(jnp.bfloat16), jnp.uint16)
    fy16 = lax.bitcast_convert_type(fyi.astype(jnp.bfloat16), jnp.uint16)
    word = (fy16.astype(jnp.uint32) << 16) | fx16.astype(jnp.uint32)
    tab = lax.bitcast_convert_type(word, jnp.float32)

    part = _temporal_sc(flow_x.reshape(B * P * N), flow_y.reshape(B * P * N), tab)

    d_dx = float(H * (W - 1))
    d_dy = float((H - 1) * W)
    d_di = float((H - 1) * (W - 1))
    per_b = (sp[:, :, 0].sum(1) / d_dx + sp[:, :, 1].sum(1) / d_dy
             + sp[:, :, 2].sum(1) / d_di + sp[:, :, 3].sum(1) / d_di) / P
    spatial = SPAT_W * jnp.sum(per_b / 4.0)

    q = part.sum(axis=(0, 3))
    dt = q[:, 0].reshape(B, P - 1)
    mm = q[:, 1].reshape(B, P - 1)
    per_b_t = (dt / (mm + 1e-9)).sum(axis=1) / (P - 1)
    temporal = TEMP_W * jnp.sum(per_b_t)

    return spatial + temporal
